# Initial kernel scaffold; baseline (speedup 1.0000x reference)
import jax
import jax.numpy as jnp
from jax.experimental import pallas as pl


def kernel(x, overs, balls_in_over, over_table, ball_table):
    raise NotImplementedError("write your pallas kernel here")


# speedup vs baseline: 4.7707x; 4.7707x over previous
import functools

import jax
import jax.numpy as jnp
from jax import lax
from jax.experimental import pallas as pl
from jax.experimental.pallas import tpu as pltpu
from jax.experimental.pallas import tpu_sc as plsc

H = 128
HH = H // 2
NC = 2
NS = 16
NW = NC * NS
LANES = 16

B, L = 4096, 200
BL = B * L
PER_W = BL // NW
P = 128
CHUNKS = PER_W // P


def _sc_kernel_body(x_hbm, ov_hbm, bl_hbm, comb_hbm, out_hbm,
                    xbuf, rows, ovidx, blidx, cidx, sem0):
    wid = lax.axis_index("s") * NC + lax.axis_index("c")

    def chunk_body(g, carry):
        base = wid * PER_W + g * P

        pltpu.sync_copy(ov_hbm.at[pl.ds(base, P)], ovidx)
        pltpu.sync_copy(bl_hbm.at[pl.ds(base, P)], blidx)
        pltpu.sync_copy(x_hbm.at[pl.ds(base * H, P * H)], xbuf)

        def idx_body(v, carry2):
            o = v * LANES
            cidx[pl.ds(o, LANES)] = (
                ovidx[pl.ds(o, LANES)] * 6 + blidx[pl.ds(o, LANES)]
            )
            return carry2

        lax.fori_loop(0, P // LANES, idx_body, 0)

        pltpu.async_copy(comb_hbm.at[cidx], rows, sem0).wait()

        def pos_body(p, carry2):
            for j in range(H // LANES):
                o = p * H + j * LANES
                xbuf[pl.ds(o, LANES)] = (
                    xbuf[pl.ds(o, LANES)] + rows[p, pl.ds(j * LANES, LANES)]
                )
            return carry2

        lax.fori_loop(0, P, pos_body, 0)

        pltpu.sync_copy(xbuf, out_hbm.at[pl.ds(base * H, P * H)])
        return carry

    lax.fori_loop(0, CHUNKS, chunk_body, 0)


@jax.jit
def _run(x_flat, ov_flat, bl_flat, comb_table):
    mesh = plsc.VectorSubcoreMesh(core_axis_name="c", subcore_axis_name="s")
    k = functools.partial(
        pl.kernel,
        mesh=mesh,
        out_type=jax.ShapeDtypeStruct((BL * H,), jnp.float32),
        scratch_types=[
            pltpu.VMEM((P * H,), jnp.float32),
            pltpu.VMEM((P, H), jnp.float32),
            pltpu.VMEM((P,), jnp.int32),
            pltpu.VMEM((P,), jnp.int32),
            pltpu.VMEM((P,), jnp.int32),
            pltpu.SemaphoreType.DMA,
        ],
    )(_sc_kernel_body)
    return k(x_flat, ov_flat, bl_flat, comb_table)


def kernel(x, overs, balls_in_over, over_table, ball_table):
    x_flat = x.reshape(BL * H)
    ov_flat = overs.reshape(BL).astype(jnp.int32)
    bl_flat = balls_in_over.reshape(BL).astype(jnp.int32)
    comb = jnp.concatenate(
        [jnp.repeat(over_table, 6, axis=0),
         jnp.tile(ball_table, (over_table.shape[0], 1))],
        axis=-1,
    )
    out = _run(x_flat, ov_flat, bl_flat, comb)
    return out.reshape(B, L, H)


# speedup vs baseline: 6.7037x; 1.4052x over previous
import functools

import jax
import jax.numpy as jnp
from jax import lax
from jax.experimental import pallas as pl
from jax.experimental.pallas import tpu as pltpu
from jax.experimental.pallas import tpu_sc as plsc

H = 128
HH = H // 2
NC = 2
NS = 16
NW = NC * NS
LANES = 16

B, L = 4096, 200
BL = B * L
PER_W = BL // NW
P = 128
CHUNKS = PER_W // P


def _sc_kernel_body(x_hbm, ov_hbm, bl_hbm, comb_hbm, out_hbm,
                    xbuf, rows, ovidx, blidx, cidx, comb_v, sem0):
    wid = lax.axis_index("s") * NC + lax.axis_index("c")

    @pl.when(lax.axis_index("s") == 0)
    def _copy_table():
        pltpu.sync_copy(comb_hbm, comb_v)

    plsc.subcore_barrier()

    def chunk_body(g, carry):
        base = wid * PER_W + g * P

        pltpu.sync_copy(ov_hbm.at[pl.ds(base, P)], ovidx)
        pltpu.sync_copy(bl_hbm.at[pl.ds(base, P)], blidx)
        pltpu.sync_copy(x_hbm.at[pl.ds(base * H, P * H)], xbuf)

        def idx_body(v, carry2):
            o = v * LANES
            cidx[pl.ds(o, LANES)] = (
                ovidx[pl.ds(o, LANES)] * 6 + blidx[pl.ds(o, LANES)]
            )
            return carry2

        lax.fori_loop(0, P // LANES, idx_body, 0)

        pltpu.async_copy(comb_v.at[cidx], rows, sem0).wait()

        def pos_body(p, carry2):
            for j in range(H // LANES):
                o = p * H + j * LANES
                xbuf[pl.ds(o, LANES)] = (
                    xbuf[pl.ds(o, LANES)] + rows[p, pl.ds(j * LANES, LANES)]
                )
            return carry2

        lax.fori_loop(0, P, pos_body, 0)

        pltpu.sync_copy(xbuf, out_hbm.at[pl.ds(base * H, P * H)])
        return carry

    lax.fori_loop(0, CHUNKS, chunk_body, 0)


@jax.jit
def _run(x_flat, ov_flat, bl_flat, comb_table):
    mesh = plsc.VectorSubcoreMesh(core_axis_name="c", subcore_axis_name="s")
    k = functools.partial(
        pl.kernel,
        mesh=mesh,
        out_type=jax.ShapeDtypeStruct((BL * H,), jnp.float32),
        scratch_types=[
            pltpu.VMEM((P * H,), jnp.float32),
            pltpu.VMEM((P, H), jnp.float32),
            pltpu.VMEM((P,), jnp.int32),
            pltpu.VMEM((P,), jnp.int32),
            pltpu.VMEM((P,), jnp.int32),
            pltpu.VMEM_SHARED((120, H), jnp.float32),
            pltpu.SemaphoreType.DMA,
        ],
    )(_sc_kernel_body)
    return k(x_flat, ov_flat, bl_flat, comb_table)


def kernel(x, overs, balls_in_over, over_table, ball_table):
    x_flat = x.reshape(BL * H)
    ov_flat = overs.reshape(BL).astype(jnp.int32)
    bl_flat = balls_in_over.reshape(BL).astype(jnp.int32)
    comb = jnp.concatenate(
        [jnp.repeat(over_table, 6, axis=0),
         jnp.tile(ball_table, (over_table.shape[0], 1))],
        axis=-1,
    )
    out = _run(x_flat, ov_flat, bl_flat, comb)
    return out.reshape(B, L, H)


# speedup vs baseline: 16.1072x; 2.4027x over previous
import functools

import jax
import jax.numpy as jnp
from jax import lax
from jax.experimental import pallas as pl
from jax.experimental.pallas import tpu as pltpu
from jax.experimental.pallas import tpu_sc as plsc

H = 128
HH = H // 2
NC = 2
NS = 16
NW = NC * NS
LANES = 16

B, L = 4096, 200
BL = B * L
PER_W = BL // NW
P = 128
CHUNKS = PER_W // P
NCOMB = 120


def _sc_kernel_body(x_hbm, ov_hbm, bl_hbm, comb_hbm, out_hbm,
                    xbuf0, xbuf1, rows0, rows1, ovidx0, ovidx1,
                    blidx0, blidx1, cidx0, cidx1, comb_v,
                    sx0, sx1, si0, si1, so0, so1, sg):
    wid = lax.axis_index("s") * NC + lax.axis_index("c")
    w0 = wid * PER_W

    xbuf = (xbuf0, xbuf1)
    rows = (rows0, rows1)
    ovidx = (ovidx0, ovidx1)
    blidx = (blidx0, blidx1)
    cidx = (cidx0, cidx1)
    sx = (sx0, sx1)
    si = (si0, si1)
    so = (so0, so1)

    @pl.when(lax.axis_index("s") == 0)
    def _copy_table():
        pltpu.sync_copy(comb_hbm, comb_v)

    plsc.subcore_barrier()

    def fire_idx(g, b):
        base = w0 + g * P
        pltpu.async_copy(ov_hbm.at[pl.ds(base, P)], ovidx[b], si[b])
        pltpu.async_copy(bl_hbm.at[pl.ds(base, P)], blidx[b], si[b])

    def wait_idx(g, b):
        base = w0 + g * P
        pltpu.make_async_copy(ov_hbm.at[pl.ds(base, P)], ovidx[b], si[b]).wait()
        pltpu.make_async_copy(bl_hbm.at[pl.ds(base, P)], blidx[b], si[b]).wait()

    def fire_x(g, b):
        base = w0 + g * P
        pltpu.async_copy(x_hbm.at[pl.ds(base * H, P * H)], xbuf[b], sx[b])

    def wait_x(g, b):
        base = w0 + g * P
        pltpu.make_async_copy(
            x_hbm.at[pl.ds(base * H, P * H)], xbuf[b], sx[b]).wait()

    def fire_out(g, b):
        base = w0 + g * P
        pltpu.async_copy(xbuf[b], out_hbm.at[pl.ds(base * H, P * H)], so[b])

    def wait_out(g, b):
        base = w0 + g * P
        pltpu.make_async_copy(
            xbuf[b], out_hbm.at[pl.ds(base * H, P * H)], so[b]).wait()

    fire_idx(0, 0)
    fire_idx(1, 1)
    fire_x(0, 0)

    def pair_body(h, carry):
        for b in range(2):
            g = h * 2 + b
            b1 = 1 - b

            wait_idx(g, b)

            def idx_body(v, c2):
                o = v * LANES
                cidx[b][pl.ds(o, LANES)] = (
                    ovidx[b][pl.ds(o, LANES)] * 6 + blidx[b][pl.ds(o, LANES)]
                )
                return c2

            lax.fori_loop(0, P // LANES, idx_body, 0)
            gather = pltpu.async_copy(comb_v.at[cidx[b]], rows[b], sg)

            @pl.when(g + 2 < CHUNKS)
            def _pf_idx():
                fire_idx(g + 2, b)

            @pl.when(g + 1 < CHUNKS)
            def _pf_x():
                @pl.when(g >= 1)
                def _drain_prev_out():
                    wait_out(g - 1, b1)

                fire_x(g + 1, b1)

            wait_x(g, b)
            gather.wait()

            def pos_body(p, c2):
                for j in range(H // LANES):
                    o = p * H + j * LANES
                    xbuf[b][pl.ds(o, LANES)] = (
                        xbuf[b][pl.ds(o, LANES)]
                        + rows[b][p, pl.ds(j * LANES, LANES)]
                    )
                return c2

            lax.fori_loop(0, P, pos_body, 0)
            fire_out(g, b)
        return carry

    lax.fori_loop(0, CHUNKS // 2, pair_body, 0)

    wait_out(CHUNKS - 2, 0)
    wait_out(CHUNKS - 1, 1)


@jax.jit
def _run(x_flat, ov_flat, bl_flat, comb_table):
    mesh = plsc.VectorSubcoreMesh(core_axis_name="c", subcore_axis_name="s")
    k = functools.partial(
        pl.kernel,
        mesh=mesh,
        out_type=jax.ShapeDtypeStruct((BL * H,), jnp.float32),
        scratch_types=[
            pltpu.VMEM((P * H,), jnp.float32),
            pltpu.VMEM((P * H,), jnp.float32),
            pltpu.VMEM((P, H), jnp.float32),
            pltpu.VMEM((P, H), jnp.float32),
            pltpu.VMEM((P,), jnp.int32),
            pltpu.VMEM((P,), jnp.int32),
            pltpu.VMEM((P,), jnp.int32),
            pltpu.VMEM((P,), jnp.int32),
            pltpu.VMEM((P,), jnp.int32),
            pltpu.VMEM((P,), jnp.int32),
            pltpu.VMEM_SHARED((NCOMB, H), jnp.float32),
            pltpu.SemaphoreType.DMA,
            pltpu.SemaphoreType.DMA,
            pltpu.SemaphoreType.DMA,
            pltpu.SemaphoreType.DMA,
            pltpu.SemaphoreType.DMA,
            pltpu.SemaphoreType.DMA,
            pltpu.SemaphoreType.DMA,
        ],
    )(_sc_kernel_body)
    return k(x_flat, ov_flat, bl_flat, comb_table)


def kernel(x, overs, balls_in_over, over_table, ball_table):
    x_flat = x.reshape(BL * H)
    ov_flat = overs.reshape(BL).astype(jnp.int32)
    bl_flat = balls_in_over.reshape(BL).astype(jnp.int32)
    comb = jnp.concatenate(
        [jnp.repeat(over_table, 6, axis=0),
         jnp.tile(ball_table, (over_table.shape[0], 1))],
        axis=-1,
    )
    out = _run(x_flat, ov_flat, bl_flat, comb)
    return out.reshape(B, L, H)
